# baseline (speedup 1.0000x reference)
import functools

import jax
import jax.numpy as jnp
from jax import lax
from jax.experimental import pallas as pl
from jax.experimental.pallas import tpu as pltpu
from jax.experimental.pallas import tpu_sc as plsc

BATCH = 4096
SEQ = 200
EMBED_DIM = 64
TOTAL = BATCH * SEQ

NC = 2
NS = 16
NW = NC * NS

PER_W = TOTAL // NW
CHUNK = 1024
K = 1
GROUP = K * CHUNK
N_GROUPS = PER_W // GROUP
N_CHUNKS = PER_W // CHUNK


def _make_gather():
  mesh = plsc.VectorSubcoreMesh(core_axis_name="c", subcore_axis_name="s")

  @functools.partial(
      pl.kernel,
      mesh=mesh,
      out_type=jax.ShapeDtypeStruct((TOTAL, EMBED_DIM), jnp.float32),
      compiler_params=pltpu.CompilerParams(use_tc_tiling_on_sc=False),
      scratch_types=[
          pltpu.VMEM((N_CHUNKS, CHUNK), jnp.int32),
          pltpu.VMEM((GROUP, EMBED_DIM), jnp.float32),
          pltpu.SemaphoreType.DMA,
      ],
  )
  def gather_kernel(idx_hbm, table_hbm, out_hbm, idx_v, rows_v, sem):
    wid = lax.axis_index("s") * NC + lax.axis_index("c")
    base = wid * PER_W

    pltpu.sync_copy(idx_hbm.at[wid], idx_v)

    def body(g, carry):
      copies = []
      for j in range(K):
        cp = pltpu.async_copy(
            table_hbm.at[idx_v.at[g * K + j]],
            rows_v.at[pl.ds(j * CHUNK, CHUNK)],
            sem,
        )
        copies.append(cp)
      for cp in copies:
        cp.wait()
      pltpu.sync_copy(rows_v, out_hbm.at[pl.ds(base + g * GROUP, GROUP)])
      return carry

    lax.fori_loop(0, N_GROUPS, body, 0, unroll=False)

  return gather_kernel


_gather = _make_gather()


@jax.jit
def kernel(indices, table):
  idx = indices.reshape(NW, N_CHUNKS, CHUNK)
  out = _gather(idx, table)
  return out.reshape(BATCH, SEQ, EMBED_DIM)


# speedup vs baseline: 1.3319x; 1.3319x over previous
import functools

import jax
import jax.numpy as jnp
from jax import lax
from jax.experimental import pallas as pl
from jax.experimental.pallas import tpu as pltpu
from jax.experimental.pallas import tpu_sc as plsc

BATCH = 4096
SEQ = 200
EMBED_DIM = 64
TOTAL = BATCH * SEQ
VOCAB = 1000000

NC = 2
NS = 16
NW = NC * NS

PER_W = TOTAL // NW
CHUNK = 128
K = 8
GROUP = K * CHUNK
N_GROUPS = PER_W // GROUP
N_CHUNKS = PER_W // CHUNK


def _make_gather():
  mesh = plsc.VectorSubcoreMesh(core_axis_name="c", subcore_axis_name="s")

  @functools.partial(
      pl.kernel,
      mesh=mesh,
      out_type=jax.ShapeDtypeStruct((TOTAL, 128), jnp.float32),
      compiler_params=pltpu.CompilerParams(use_tc_tiling_on_sc=False),
      scratch_types=[
          pltpu.VMEM((N_CHUNKS, CHUNK), jnp.int32),
          pltpu.VMEM((GROUP, EMBED_DIM), jnp.float32),
          pltpu.SemaphoreType.DMA,
      ],
  )
  def gather_kernel(idx_hbm, table_hbm, out_hbm, idx_v, rows_v, sem):
    wid = lax.axis_index("s") * NC + lax.axis_index("c")
    base = wid * PER_W

    pltpu.sync_copy(idx_hbm.at[wid], idx_v)

    def body(g, carry):
      copies = []
      for j in range(K):
        cp = pltpu.async_copy(
            table_hbm.at[idx_v.at[g * K + j]],
            rows_v.at[pl.ds(j * CHUNK, CHUNK)],
            sem,
        )
        copies.append(cp)
      for cp in copies:
        cp.wait()
      pltpu.sync_copy(
          rows_v,
          out_hbm.at[pl.ds(base + g * GROUP, GROUP), pl.ds(0, EMBED_DIM)],
      )
      return carry

    lax.fori_loop(0, N_GROUPS, body, 0, unroll=False)

  return gather_kernel


_gather = _make_gather()


@jax.jit
def kernel(indices, table):
  idx = indices.reshape(NW, N_CHUNKS, CHUNK)
  t128 = lax.optimization_barrier(table.reshape(VOCAB // 2, 128))
  t64 = t128.reshape(VOCAB, EMBED_DIM)
  out = _gather(idx, t64)
  return out.reshape(BATCH, SEQ, 128)[:, :, :EMBED_DIM]


# speedup vs baseline: 1.3443x; 1.0093x over previous
import functools

import jax
import jax.numpy as jnp
from jax import lax
from jax.experimental import pallas as pl
from jax.experimental.pallas import tpu as pltpu
from jax.experimental.pallas import tpu_sc as plsc

BATCH = 4096
SEQ = 200
EMBED_DIM = 64
TOTAL = BATCH * SEQ
VOCAB = 1000000
TABLE_ROWS = 2 * VOCAB

NC = 2
NS = 16
NW = NC * NS

PER_W = TOTAL // NW
CHUNK = 128
K = 4
GROUP = K * CHUNK
N_GROUPS = PER_W // GROUP
N_CHUNKS = PER_W // CHUNK


def _make_gather():
  mesh = plsc.VectorSubcoreMesh(core_axis_name="c", subcore_axis_name="s")

  @functools.partial(
      pl.kernel,
      mesh=mesh,
      out_type=jax.ShapeDtypeStruct((TOTAL, 128), jnp.float32),
      compiler_params=pltpu.CompilerParams(use_tc_tiling_on_sc=False),
      scratch_types=[
          pltpu.VMEM((N_CHUNKS, CHUNK), jnp.int32),
          pltpu.VMEM((2, GROUP, EMBED_DIM), jnp.float32),
          pltpu.SemaphoreType.DMA,
          pltpu.SemaphoreType.DMA,
      ],
  )
  def gather_kernel(idx_hbm, table_hbm, out_hbm, idx_v, rows_v, sem, osem):
    wid = lax.axis_index("s") * NC + lax.axis_index("c")
    base = wid * PER_W

    pltpu.sync_copy(idx_hbm.at[wid], idx_v)

    def out_slice(g):
      return out_hbm.at[pl.ds(base + g * GROUP, GROUP), pl.ds(0, EMBED_DIM)]

    def body(g, carry):
      par = lax.rem(g, 2)
      @pl.when(g >= 2)
      def _():
        pltpu.make_async_copy(rows_v.at[par], out_slice(g), osem).wait()

      copies = []
      for j in range(K):
        cp = pltpu.async_copy(
            table_hbm.at[idx_v.at[g * K + j]],
            rows_v.at[par, pl.ds(j * CHUNK, CHUNK)],
            sem,
        )
        copies.append(cp)
      for cp in copies:
        cp.wait()
      pltpu.async_copy(rows_v.at[par], out_slice(g), osem)
      return carry

    lax.fori_loop(0, N_GROUPS, body, 0, unroll=False)
    for g in (N_GROUPS - 2, N_GROUPS - 1):
      pltpu.make_async_copy(
          rows_v.at[lax.rem(g, 2)], out_slice(g), osem
      ).wait()

  return gather_kernel


_gather = _make_gather()


@jax.jit
def kernel(indices, table):
  idx = indices.reshape(NW, N_CHUNKS, CHUNK)
  t128 = lax.optimization_barrier(table.reshape(VOCAB // 2, 128))
  t64 = t128.reshape(VOCAB, EMBED_DIM)
  out = _gather(idx, t64)
  return out.reshape(BATCH, SEQ, 128)[:, :, :EMBED_DIM]
